# baseline (speedup 1.0000x reference)
import jax
import jax.numpy as jnp
from jax.experimental import pallas as pl

DEPTH = 1000
ROWS = 1024 * 26
BLOCK_ROWS = 1024


def _onehot_block(idx_ref, out_ref):
    idx = idx_ref[...]
    col = jax.lax.broadcasted_iota(jnp.int32, (BLOCK_ROWS, DEPTH), 1)
    out_ref[...] = (col == idx).astype(jnp.float32)


def kernel(indices):
    flat = indices.reshape(ROWS, 1)
    out = pl.pallas_call(
        _onehot_block,
        grid=(ROWS // BLOCK_ROWS,),
        in_specs=[pl.BlockSpec((BLOCK_ROWS, 1), lambda i: (i, 0))],
        out_specs=pl.BlockSpec((BLOCK_ROWS, DEPTH), lambda i: (i, 0)),
        out_shape=jax.ShapeDtypeStruct((ROWS, DEPTH), jnp.float32),
    )(flat)
    return out.reshape(indices.shape[0], indices.shape[1], DEPTH)


# speedup vs baseline: 1.4499x; 1.4499x over previous
import jax
import jax.numpy as jnp
from jax.experimental import pallas as pl

DEPTH = 1000
BATCH = 1024
FEATS = 26
BLOCK_B = 64


def _onehot_block(idx_ref, out_ref):
    idx = idx_ref[...]
    col = jax.lax.broadcasted_iota(jnp.int32, (BLOCK_B, FEATS, DEPTH), 2)
    out_ref[...] = (col == idx[:, :, None]).astype(jnp.float32)


def kernel(indices):
    return pl.pallas_call(
        _onehot_block,
        grid=(BATCH // BLOCK_B,),
        in_specs=[pl.BlockSpec((BLOCK_B, FEATS), lambda i: (i, 0))],
        out_specs=pl.BlockSpec((BLOCK_B, FEATS, DEPTH), lambda i: (i, 0, 0)),
        out_shape=jax.ShapeDtypeStruct((BATCH, FEATS, DEPTH), jnp.float32),
    )(indices)
